# baseline (speedup 1.0000x reference)
import functools

import jax
import jax.numpy as jnp
from jax import lax
from jax.experimental import pallas as pl
from jax.experimental.pallas import tpu as pltpu
from jax.experimental.pallas import tpu_sc as plsc

N_ITEMS = 100000
EMBED_DIM = 64
BATCH = 4096
HIST = 50
TOPK = 21

L = 16
NC, NS = 2, 16
NW = NC * NS
ROWS_W = BATCH // NW
NBAT = ROWS_W // L

BLK = 128
V_PAD = 100352
NB = V_PAD // BLK
NGRP = NB // L
NSEL = 24
NCH = (NSEL * L) // 128
H_PAD = 64

RB = 512
CB = 1024
NI = BATCH // RB
NJ = V_PAD // CB

NEG = float("-inf")
BIG = 2**30

_mesh = plsc.VectorSubcoreMesh(core_axis_name="c", subcore_axis_name="s")


def _splat_i(x):
    return jnp.full((L,), x, jnp.int32)


def _splat_f(x):
    return jnp.full((L,), x, jnp.float32)


def _tree_argmax(pairs):
    while len(pairs) > 1:
        nxt = []
        for i in range(0, len(pairs) - 1, 2):
            (v1, i1), (v2, i2) = pairs[i], pairs[i + 1]
            upd = v2 > v1
            nxt.append((jnp.maximum(v1, v2), jnp.where(upd, i2, i1)))
        if len(pairs) % 2:
            nxt.append(pairs[-1])
        pairs = nxt
    return pairs[0]


NQB = 4

@functools.partial(
    pl.kernel,
    mesh=_mesh,
    compiler_params=pltpu.CompilerParams(needs_layout_passes=False),
    out_type=jax.ShapeDtypeStruct((BATCH * EMBED_DIM,), jnp.float32),
    scratch_types=[
        pltpu.VMEM((ROWS_W // 2, 128), jnp.int32),
        pltpu.VMEM((ROWS_W,), jnp.int32),
        [pltpu.VMEM((128, 2 * EMBED_DIM), jnp.float32) for _ in range(NQB)],
        pltpu.VMEM((ROWS_W * EMBED_DIM,), jnp.float32),
        [pltpu.SemaphoreType.DMA for _ in range(NQB)],
    ],
)
def _query_k(seq_hbm, len_hbm, emb_hbm, out_hbm,
             seq_v, len_v, rbs, q_v, sems):
    wid = lax.axis_index("s") * NC + lax.axis_index("c")
    base = wid * ROWS_W
    iota = lax.iota(jnp.int32, L)
    pltpu.sync_copy(seq_hbm.at[pl.ds(wid * (ROWS_W // 2), ROWS_W // 2)], seq_v)
    pltpu.sync_copy(len_hbm.at[pl.ds(base, ROWS_W)], len_v)

    NCHK = ROWS_W // 2

    def fire(c, u):
        pltpu.async_copy(emb_hbm.at[seq_v.at[c]], rbs[u], sems[u])

    def process(c, u):
        rb = rbs[u]
        for r in range(2):
            b = c * 2 + r
            lenb = plsc.load_gather(len_v, [_splat_i(b)])
            len_f = lenb.astype(jnp.float32)
            acc = [jnp.zeros((L,), jnp.float32) for _ in range(4)]
            for t in range(H_PAD):
                wt = jnp.where(t < lenb, 1.0, 0.0).astype(jnp.float32)
                for j in range(4):
                    acc[j] = acc[j] + rb[r * H_PAD + t, pl.ds(j * L, L)] * wt
            addr0 = _splat_i(b * EMBED_DIM) + iota
            for j in range(4):
                plsc.store_scatter(q_v, [addr0 + j * L], acc[j] / len_f)

    for u in range(NQB):
        fire(u, u)

    def chunk_body(c, _):
        for u in range(NQB):
            @pl.when(c % NQB == u)
            def _():
                pltpu.make_async_copy(
                    emb_hbm.at[pl.ds(0, 128)], rbs[u], sems[u]).wait()
                process(c, u)

                @pl.when(c + NQB < NCHK)
                def _():
                    fire(c + NQB, u)

        return 0

    lax.fori_loop(0, NCHK, chunk_body, 0)
    pltpu.sync_copy(q_v, out_hbm.at[pl.ds(base * EMBED_DIM, ROWS_W * EMBED_DIM)])


def _score_body(q_ref, e_ref, s_ref, bm_ref):
    j = pl.program_id(0)
    s = lax.dot_general(
        q_ref[...], e_ref[...], (((1,), (1,)), ((), ())),
        preferred_element_type=jnp.float32,
    )
    def emit(sv):
        s_ref[...] = sv
        bms = [jnp.max(sv[:, sub * BLK:(sub + 1) * BLK], axis=1)
               for sub in range(CB // BLK)]
        st = jnp.stack(bms, axis=1)
        bm_ref[...] = jnp.transpose(
            st.reshape(RB // L, L, CB // BLK), (0, 2, 1))

    @pl.when(j < NJ - 1)
    def _():
        emit(s)

    @pl.when(j == NJ - 1)
    def _():
        cols = (NJ - 1) * CB + lax.broadcasted_iota(jnp.int32, (RB, CB), 1)
        emit(jnp.where(cols < N_ITEMS, s, NEG))


def _scores_call(query, emb_pad):
    return pl.pallas_call(
        _score_body,
        grid=(NJ, NI),
        in_specs=[
            pl.BlockSpec((RB, EMBED_DIM), lambda j, i: (i, 0)),
            pl.BlockSpec((CB, EMBED_DIM), lambda j, i: (j, 0)),
        ],
        out_specs=[
            pl.BlockSpec((RB, CB), lambda j, i: (i, j)),
            pl.BlockSpec((RB // L, CB // BLK, L), lambda j, i: (i, j, 0)),
        ],
        out_shape=[
            jax.ShapeDtypeStruct((BATCH, V_PAD), jnp.float32),
            jax.ShapeDtypeStruct((BATCH // L, NB, L), jnp.float32),
        ],
    )(query, emb_pad)


@functools.partial(
    pl.kernel,
    mesh=_mesh,
    compiler_params=pltpu.CompilerParams(needs_layout_passes=False),
    out_type=(
        jax.ShapeDtypeStruct((BATCH // L, NSEL * L), jnp.float32),
        jax.ShapeDtypeStruct((BATCH // L, NSEL * L), jnp.int32),
    ),
    scratch_types=[
        pltpu.VMEM((NB // 8, 8 * L), jnp.float32),
        pltpu.VMEM((NGRP * L,), jnp.float32),
        pltpu.VMEM((NGRP * L,), jnp.int32),
        pltpu.VMEM((NSEL * L,), jnp.float32),
        pltpu.VMEM((NSEL * L,), jnp.int32),
        pltpu.VMEM((NCH, 128), jnp.int32),
        pltpu.VMEM((NSEL * L, BLK), jnp.float32),
        pltpu.VMEM((NSEL * L,), jnp.float32),
        pltpu.VMEM((NSEL * L,), jnp.int32),
        pltpu.SemaphoreType.DMA,
    ],
)
def _topk_k(sc_hbm, bm_hbm, vals_hbm, idxs_hbm,
            bm_v, gmax_v, gidx_v, selmax_v, selbid_v, gids_v, cand_v,
            ov_v, oi_v, sem):
    wid = lax.axis_index("s") * NC + lax.axis_index("c")
    base = wid * ROWS_W
    iota = lax.iota(jnp.int32, L)

    def bat_body(bi, _):
        r0 = base + bi * L
        rows = _splat_i(r0) + iota
        pltpu.sync_copy(bm_hbm.at[wid * NBAT + bi], bm_v)

        for q in range(NGRP):
            v, idv = _tree_argmax(
                [(bm_v[(q * L + k2) // 8, pl.ds(((q * L + k2) % 8) * L, L)],
                  _splat_i(q * L + k2)) for k2 in range(L)])
            gmax_v[pl.ds(q * L, L)] = v
            gidx_v[pl.ds(q * L, L)] = idv

        def sel_body(k, _):
            gv, gq = _tree_argmax(
                [(gmax_v[pl.ds(q * L, L)], _splat_i(q)) for q in range(NGRP)])
            lanes = gq * L + iota
            bidv = plsc.load_gather(gidx_v, [lanes])
            kv = _splat_i(k)
            plsc.store_scatter(selmax_v, [kv * L + iota], gv)
            plsc.store_scatter(selbid_v, [kv * L + iota], bidv)
            plsc.store_scatter(
                gids_v, [_splat_i(k // 8), _splat_i((k % 8) * L) + iota],
                rows * NB + bidv)
            plsc.store_scatter(
                bm_v, [bidv >> 3, ((bidv & 7) * L) + iota], _splat_f(NEG))
            gbase = gq * L
            def _bm_row(bid2):
                return plsc.load_gather(
                    bm_v, [bid2 >> 3, ((bid2 & 7) * L) + iota])
            nv, nid = _tree_argmax(
                [(_bm_row(gbase + k2), gbase + k2) for k2 in range(L)])
            plsc.store_scatter(gmax_v, [lanes], nv)
            plsc.store_scatter(gidx_v, [lanes], nid)
            return 0

        lax.fori_loop(0, NSEL, sel_body, 0)

        handles = [
            pltpu.async_copy(sc_hbm.at[gids_v.at[c]],
                             cand_v.at[pl.ds(c * 128, 128)], sem)
            for c in range(NCH)
        ]
        for h in handles:
            h.wait()

        def ext_body(k, _):
            trip = [(selmax_v[pl.ds(s * L, L)], selbid_v[pl.ds(s * L, L)],
                     _splat_i(s)) for s in range(NSEL)]
            while len(trip) > 1:
                nxt = []
                for i in range(0, len(trip) - 1, 2):
                    v1, b1, s1 = trip[i]
                    v2, b2, s2 = trip[i + 1]
                    upd = (v2 > v1) | ((v2 == v1) & (b2 < b1))
                    nxt.append((jnp.where(upd, v2, v1),
                                jnp.where(upd, b2, b1),
                                jnp.where(upd, s2, s1)))
                if len(trip) % 2:
                    nxt.append(trip[-1])
                trip = nxt
            _, bwin, swin = trip[0]
            crow = swin * L + iota
            m = _splat_f(NEG)
            m2 = _splat_f(NEG)
            boff = _splat_i(0)
            for e in range(BLK):
                v = plsc.load_gather(cand_v, [crow, _splat_i(e)])
                upd = v > m
                lo = jnp.minimum(m, v)
                m = jnp.maximum(m, v)
                m2 = jnp.maximum(m2, lo)
                boff = jnp.where(upd, e, boff)
            kv = _splat_i(k)
            plsc.store_scatter(ov_v, [kv * L + iota], m)
            plsc.store_scatter(oi_v, [kv * L + iota], bwin * BLK + boff)
            plsc.store_scatter(cand_v, [crow, boff], _splat_f(NEG))
            plsc.store_scatter(selmax_v, [swin * L + iota], m2)
            return 0

        lax.fori_loop(0, TOPK, ext_body, 0)

        gbat = wid * NBAT + bi
        pltpu.sync_copy(ov_v, vals_hbm.at[gbat])
        pltpu.sync_copy(oi_v, idxs_hbm.at[gbat])
        return 0

    lax.fori_loop(0, NBAT, bat_body, 0)


def kernel(item_seq, item_seq_len, item_embedding):
    seq = item_seq.astype(jnp.int32)
    lens = jnp.maximum(item_seq_len.astype(jnp.int32), 1)
    emb_pad = jnp.pad(item_embedding, ((0, V_PAD - N_ITEMS), (0, 0)))
    emb_sc = jnp.pad(item_embedding, ((0, 0), (0, EMBED_DIM)))
    seq_pad = jnp.pad(seq, ((0, 0), (0, H_PAD - HIST)))

    query = _query_k(seq_pad.reshape(BATCH * H_PAD // 128, 128), lens,
                     emb_sc).reshape(BATCH, EMBED_DIM)
    scores, bm = _scores_call(query, emb_pad)
    vals3, idxs3 = _topk_k(scores.reshape(BATCH * NB, BLK),
                           bm.reshape(BATCH // L, NB // 8, 8 * L))
    vals = jnp.transpose(vals3.reshape(BATCH // L, NSEL, L),
                         (0, 2, 1)).reshape(BATCH, NSEL)
    idxs = jnp.transpose(idxs3.reshape(BATCH // L, NSEL, L),
                         (0, 2, 1)).reshape(BATCH, NSEL)
    return vals[:, :TOPK], idxs[:, :TOPK]


# speedup vs baseline: 1.2279x; 1.2279x over previous
import functools

import jax
import jax.numpy as jnp
from jax import lax
from jax.experimental import pallas as pl
from jax.experimental.pallas import tpu as pltpu
from jax.experimental.pallas import tpu_sc as plsc

N_ITEMS = 100000
EMBED_DIM = 64
BATCH = 4096
HIST = 50
TOPK = 21

L = 16
NC, NS = 2, 16
NW = NC * NS
ROWS_W = BATCH // NW
NBAT = ROWS_W // L

BLK = 128
V_PAD = 100352
NB = V_PAD // BLK
NGRP = NB // L
NSEL = 24
NCH = (NSEL * L) // 128
H_PAD = 64

RB = 512
CB = 1024
NI = BATCH // RB
NJ = V_PAD // CB

NEG = float("-inf")
BIG = 2**30

_mesh = plsc.VectorSubcoreMesh(core_axis_name="c", subcore_axis_name="s")


def _splat_i(x):
    return jnp.full((L,), x, jnp.int32)


def _splat_f(x):
    return jnp.full((L,), x, jnp.float32)


def _tree_argmax(pairs):
    while len(pairs) > 1:
        nxt = []
        for i in range(0, len(pairs) - 1, 2):
            (v1, i1), (v2, i2) = pairs[i], pairs[i + 1]
            upd = v2 > v1
            nxt.append((jnp.maximum(v1, v2), jnp.where(upd, i2, i1)))
        if len(pairs) % 2:
            nxt.append(pairs[-1])
        pairs = nxt
    return pairs[0]


NQB = 4

@functools.partial(
    pl.kernel,
    mesh=_mesh,
    compiler_params=pltpu.CompilerParams(needs_layout_passes=False),
    out_type=jax.ShapeDtypeStruct((BATCH * EMBED_DIM,), jnp.float32),
    scratch_types=[
        pltpu.VMEM((ROWS_W // 2, 128), jnp.int32),
        pltpu.VMEM((ROWS_W,), jnp.int32),
        [pltpu.VMEM((128, 2 * EMBED_DIM), jnp.float32) for _ in range(NQB)],
        pltpu.VMEM((ROWS_W * EMBED_DIM,), jnp.float32),
        [pltpu.SemaphoreType.DMA for _ in range(NQB)],
    ],
)
def _query_k(seq_hbm, len_hbm, emb_hbm, out_hbm,
             seq_v, len_v, rbs, q_v, sems):
    wid = lax.axis_index("s") * NC + lax.axis_index("c")
    base = wid * ROWS_W
    iota = lax.iota(jnp.int32, L)
    pltpu.sync_copy(seq_hbm.at[pl.ds(wid * (ROWS_W // 2), ROWS_W // 2)], seq_v)
    pltpu.sync_copy(len_hbm.at[pl.ds(base, ROWS_W)], len_v)

    NCHK = ROWS_W // 2

    def fire(c, u):
        pltpu.async_copy(emb_hbm.at[seq_v.at[c]], rbs[u], sems[u])

    def process(c, u):
        rb = rbs[u]
        for r in range(2):
            b = c * 2 + r
            lenb = plsc.load_gather(len_v, [_splat_i(b)])
            len_f = lenb.astype(jnp.float32)
            acc = [jnp.zeros((L,), jnp.float32) for _ in range(4)]
            for t in range(H_PAD):
                wt = jnp.where(t < lenb, 1.0, 0.0).astype(jnp.float32)
                for j in range(4):
                    acc[j] = acc[j] + rb[r * H_PAD + t, pl.ds(j * L, L)] * wt
            addr0 = _splat_i(b * EMBED_DIM) + iota
            for j in range(4):
                plsc.store_scatter(q_v, [addr0 + j * L], acc[j] / len_f)

    for u in range(NQB):
        fire(u, u)

    def chunk_body(c, _):
        for u in range(NQB):
            @pl.when(c % NQB == u)
            def _():
                pltpu.make_async_copy(
                    emb_hbm.at[pl.ds(0, 128)], rbs[u], sems[u]).wait()
                process(c, u)

                @pl.when(c + NQB < NCHK)
                def _():
                    fire(c + NQB, u)

        return 0

    lax.fori_loop(0, NCHK, chunk_body, 0)
    pltpu.sync_copy(q_v, out_hbm.at[pl.ds(base * EMBED_DIM, ROWS_W * EMBED_DIM)])


def _score_body(q_ref, e_ref, s_ref, bm_ref):
    j = pl.program_id(0)
    s = lax.dot_general(
        q_ref[...], e_ref[...], (((1,), (1,)), ((), ())),
        preferred_element_type=jnp.float32,
    )
    def emit(sv):
        s_ref[...] = sv.reshape(RB, CB // BLK, BLK)
        bms = [jnp.max(sv[:, sub * BLK:(sub + 1) * BLK], axis=1)
               for sub in range(CB // BLK)]
        st = jnp.stack(bms, axis=1)
        bm_ref[...] = jnp.transpose(
            st.reshape(RB // L, L, CB // BLK), (0, 2, 1))

    @pl.when(j < NJ - 1)
    def _():
        emit(s)

    @pl.when(j == NJ - 1)
    def _():
        cols = (NJ - 1) * CB + lax.broadcasted_iota(jnp.int32, (RB, CB), 1)
        emit(jnp.where(cols < N_ITEMS, s, NEG))


def _scores_call(query, emb_pad):
    return pl.pallas_call(
        _score_body,
        grid=(NJ, NI),
        in_specs=[
            pl.BlockSpec((RB, EMBED_DIM), lambda j, i: (i, 0)),
            pl.BlockSpec((CB, EMBED_DIM), lambda j, i: (j, 0)),
        ],
        out_specs=[
            pl.BlockSpec((RB, CB // BLK, BLK), lambda j, i: (i, j, 0)),
            pl.BlockSpec((RB // L, CB // BLK, L), lambda j, i: (i, j, 0)),
        ],
        out_shape=[
            jax.ShapeDtypeStruct((BATCH, NB, BLK), jnp.float32),
            jax.ShapeDtypeStruct((BATCH // L, NB, L), jnp.float32),
        ],
    )(query, emb_pad)


@functools.partial(
    pl.kernel,
    mesh=_mesh,
    compiler_params=pltpu.CompilerParams(needs_layout_passes=False),
    out_type=(
        jax.ShapeDtypeStruct((BATCH // L, NSEL * L), jnp.float32),
        jax.ShapeDtypeStruct((BATCH // L, NSEL * L), jnp.int32),
    ),
    scratch_types=[
        pltpu.VMEM((NB // 8, 8 * L), jnp.float32),
        pltpu.VMEM((NGRP * L,), jnp.float32),
        pltpu.VMEM((NGRP * L,), jnp.int32),
        pltpu.VMEM((NSEL * L,), jnp.float32),
        pltpu.VMEM((NSEL * L,), jnp.int32),
        pltpu.VMEM((NCH, 128), jnp.int32),
        pltpu.VMEM((NSEL * L, BLK), jnp.float32),
        pltpu.VMEM((NSEL * L,), jnp.float32),
        pltpu.VMEM((NSEL * L,), jnp.int32),
        pltpu.SemaphoreType.DMA,
    ],
)
def _topk_k(sc_hbm, bm_hbm, vals_hbm, idxs_hbm,
            bm_v, gmax_v, gidx_v, selmax_v, selbid_v, gids_v, cand_v,
            ov_v, oi_v, sem):
    wid = lax.axis_index("s") * NC + lax.axis_index("c")
    base = wid * ROWS_W
    iota = lax.iota(jnp.int32, L)

    def bat_body(bi, _):
        r0 = base + bi * L
        rows = _splat_i(r0) + iota
        pltpu.sync_copy(bm_hbm.at[wid * NBAT + bi], bm_v)

        for q in range(NGRP):
            v, idv = _tree_argmax(
                [(bm_v[(q * L + k2) // 8, pl.ds(((q * L + k2) % 8) * L, L)],
                  _splat_i(q * L + k2)) for k2 in range(L)])
            gmax_v[pl.ds(q * L, L)] = v
            gidx_v[pl.ds(q * L, L)] = idv

        def sel_body(k, _):
            gv, gq = _tree_argmax(
                [(gmax_v[pl.ds(q * L, L)], _splat_i(q)) for q in range(NGRP)])
            lanes = gq * L + iota
            bidv = plsc.load_gather(gidx_v, [lanes])
            kv = _splat_i(k)
            plsc.store_scatter(selmax_v, [kv * L + iota], gv)
            plsc.store_scatter(selbid_v, [kv * L + iota], bidv)
            plsc.store_scatter(
                gids_v, [_splat_i(k // 8), _splat_i((k % 8) * L) + iota],
                rows * NB + bidv)
            plsc.store_scatter(
                bm_v, [bidv >> 3, ((bidv & 7) * L) + iota], _splat_f(NEG))
            gbase = gq * L
            def _bm_row(bid2):
                return plsc.load_gather(
                    bm_v, [bid2 >> 3, ((bid2 & 7) * L) + iota])
            nv, nid = _tree_argmax(
                [(_bm_row(gbase + k2), gbase + k2) for k2 in range(L)])
            plsc.store_scatter(gmax_v, [lanes], nv)
            plsc.store_scatter(gidx_v, [lanes], nid)
            return 0

        lax.fori_loop(0, NSEL, sel_body, 0)

        handles = [
            pltpu.async_copy(sc_hbm.at[gids_v.at[c]],
                             cand_v.at[pl.ds(c * 128, 128)], sem)
            for c in range(NCH)
        ]
        for h in handles:
            h.wait()

        def ext_body(k, _):
            trip = [(selmax_v[pl.ds(s * L, L)], selbid_v[pl.ds(s * L, L)],
                     _splat_i(s)) for s in range(NSEL)]
            while len(trip) > 1:
                nxt = []
                for i in range(0, len(trip) - 1, 2):
                    v1, b1, s1 = trip[i]
                    v2, b2, s2 = trip[i + 1]
                    upd = (v2 > v1) | ((v2 == v1) & (b2 < b1))
                    nxt.append((jnp.where(upd, v2, v1),
                                jnp.where(upd, b2, b1),
                                jnp.where(upd, s2, s1)))
                if len(trip) % 2:
                    nxt.append(trip[-1])
                trip = nxt
            _, bwin, swin = trip[0]
            crow = swin * L + iota
            m = _splat_f(NEG)
            m2 = _splat_f(NEG)
            boff = _splat_i(0)
            for e in range(BLK):
                v = plsc.load_gather(cand_v, [crow, _splat_i(e)])
                upd = v > m
                lo = jnp.minimum(m, v)
                m = jnp.maximum(m, v)
                m2 = jnp.maximum(m2, lo)
                boff = jnp.where(upd, e, boff)
            kv = _splat_i(k)
            plsc.store_scatter(ov_v, [kv * L + iota], m)
            plsc.store_scatter(oi_v, [kv * L + iota], bwin * BLK + boff)
            plsc.store_scatter(cand_v, [crow, boff], _splat_f(NEG))
            plsc.store_scatter(selmax_v, [swin * L + iota], m2)
            return 0

        lax.fori_loop(0, TOPK, ext_body, 0)

        gbat = wid * NBAT + bi
        pltpu.sync_copy(ov_v, vals_hbm.at[gbat])
        pltpu.sync_copy(oi_v, idxs_hbm.at[gbat])
        return 0

    lax.fori_loop(0, NBAT, bat_body, 0)


def kernel(item_seq, item_seq_len, item_embedding):
    seq = item_seq.astype(jnp.int32)
    lens = jnp.maximum(item_seq_len.astype(jnp.int32), 1)
    emb_pad = jnp.pad(item_embedding, ((0, V_PAD - N_ITEMS), (0, 0)))
    emb_sc = jnp.pad(item_embedding, ((0, 0), (0, EMBED_DIM)))
    seq_pad = jnp.pad(seq, ((0, 0), (0, H_PAD - HIST)))

    query = _query_k(seq_pad.reshape(BATCH * H_PAD // 128, 128), lens,
                     emb_sc).reshape(BATCH, EMBED_DIM)
    scores, bm = _scores_call(query, emb_pad)
    vals3, idxs3 = _topk_k(scores.reshape(BATCH * NB, BLK),
                           bm.reshape(BATCH // L, NB // 8, 8 * L))
    vals = jnp.transpose(vals3.reshape(BATCH // L, NSEL, L),
                         (0, 2, 1)).reshape(BATCH, NSEL)
    idxs = jnp.transpose(idxs3.reshape(BATCH // L, NSEL, L),
                         (0, 2, 1)).reshape(BATCH, NSEL)
    return vals[:, :TOPK], idxs[:, :TOPK]
